# baseline (speedup 1.0000x reference)
import functools

import jax
import jax.numpy as jnp
from jax import lax
from jax.experimental import pallas as pl
from jax.experimental.pallas import tpu as pltpu
from jax.experimental.pallas import tpu_sc as plsc


NUM_ROWS = 100000
EMBED = 64
FLAT = NUM_ROWS * EMBED
NUM_CORES = 2
NUM_SUBCORES = 16
NUM_WORKERS = NUM_CORES * NUM_SUBCORES
PER_CORE = FLAT // NUM_CORES
NBUF = 3
CHUNK = 400_000
NUM_CHUNKS = PER_CORE // CHUNK


@functools.partial(
    pl.kernel,
    mesh=plsc.VectorSubcoreMesh(core_axis_name="c", subcore_axis_name="s"),
    out_type=jax.ShapeDtypeStruct((FLAT,), jnp.float32),
    scratch_types=[
        pltpu.VMEM_SHARED((CHUNK,), jnp.float32),
        pltpu.VMEM_SHARED((CHUNK,), jnp.float32),
        pltpu.VMEM_SHARED((CHUNK,), jnp.float32),
        pltpu.SemaphoreType.DMA,
        pltpu.SemaphoreType.DMA,
        pltpu.SemaphoreType.DMA,
        pltpu.SemaphoreType.DMA,
        pltpu.SemaphoreType.DMA,
        pltpu.SemaphoreType.DMA,
    ],
)
def _sc_copy(in_hbm, out_hbm, sh0, sh1, sh2, is0, is1, is2, os0, os1, os2):
    c = lax.axis_index("c")
    s = lax.axis_index("s")
    base = c * PER_CORE
    bufs = (sh0, sh1, sh2)
    isems = (is0, is1, is2)
    osems = (os0, os1, os2)

    @pl.when(s == 0)
    def _driver():
        in_cp = []
        out_cp = []
        for i in range(NUM_CHUNKS):
            sl = pl.ds(base + i * CHUNK, CHUNK)
            b = bufs[i % NBUF]
            in_cp.append(pltpu.make_async_copy(in_hbm.at[sl], b, isems[i % NBUF]))
            out_cp.append(pltpu.make_async_copy(b, out_hbm.at[sl], osems[i % NBUF]))

        for i in range(NBUF):
            in_cp[i].start()
        for i in range(NUM_CHUNKS):
            in_cp[i].wait()
            out_cp[i].start()
            if i + NBUF < NUM_CHUNKS:
                out_cp[i].wait()
                in_cp[i + NBUF].start()
        for i in range(NUM_CHUNKS - NBUF, NUM_CHUNKS):
            out_cp[i].wait()


def kernel(x, W):
    del x
    flat = W.T.reshape(-1)
    out_flat = _sc_copy(flat)
    return out_flat.reshape(EMBED, NUM_ROWS).T


# speedup vs baseline: 2.5602x; 2.5602x over previous
import functools

import jax
import jax.numpy as jnp
from jax import lax
from jax.experimental import pallas as pl
from jax.experimental.pallas import tpu as pltpu
from jax.experimental.pallas import tpu_sc as plsc


NUM_ROWS = 100000
EMBED = 64
FLAT = NUM_ROWS * EMBED
NUM_CORES = 2
NUM_SUBCORES = 16
NUM_WORKERS = NUM_CORES * NUM_SUBCORES
ROWS_PER_CORE = EMBED // NUM_CORES
NBUF = 2
STRIPE = 8
NUM_CHUNKS = ROWS_PER_CORE // STRIPE


@functools.partial(
    pl.kernel,
    mesh=plsc.VectorSubcoreMesh(core_axis_name="c", subcore_axis_name="s"),
    out_type=jax.ShapeDtypeStruct((EMBED, NUM_ROWS), jnp.float32),
    scratch_types=[
        pltpu.VMEM_SHARED((STRIPE, NUM_ROWS), jnp.float32),
        pltpu.VMEM_SHARED((STRIPE, NUM_ROWS), jnp.float32),
        pltpu.SemaphoreType.DMA,
        pltpu.SemaphoreType.DMA,
        pltpu.SemaphoreType.DMA,
        pltpu.SemaphoreType.DMA,
    ],
    compiler_params=pltpu.CompilerParams(use_tc_tiling_on_sc=True),
)
def _sc_copy(in_hbm, out_hbm, sh0, sh1, is0, is1, os0, os1):
    c = lax.axis_index("c")
    s = lax.axis_index("s")
    base = c * ROWS_PER_CORE
    bufs = (sh0, sh1)
    isems = (is0, is1)
    osems = (os0, os1)

    @pl.when(s == 0)
    def _driver():
        in_cp = []
        out_cp = []
        for i in range(NUM_CHUNKS):
            sl = pl.ds(base + i * STRIPE, STRIPE)
            b = bufs[i % NBUF]
            in_cp.append(
                pltpu.make_async_copy(in_hbm.at[sl, :], b, isems[i % NBUF])
            )
            out_cp.append(
                pltpu.make_async_copy(b, out_hbm.at[sl, :], osems[i % NBUF])
            )

        for i in range(NBUF):
            in_cp[i].start()
        for i in range(NUM_CHUNKS):
            in_cp[i].wait()
            out_cp[i].start()
            if i + NBUF < NUM_CHUNKS:
                out_cp[i].wait()
                in_cp[i + NBUF].start()
        for i in range(NUM_CHUNKS - NBUF, NUM_CHUNKS):
            out_cp[i].wait()


def kernel(x, W):
    del x
    out_t = _sc_copy(W.T)
    return out_t.T


# speedup vs baseline: 6.6340x; 2.5912x over previous
import jax
import jax.numpy as jnp
from jax.experimental import pallas as pl
from jax.experimental.pallas import tpu as pltpu


NUM_ROWS = 100000
EMBED = 64
NSTRIPE = 8
STRIPE = EMBED // NSTRIPE


def _dma_copy_kernel(w_hbm, o_hbm, *scratch):
    bufs = scratch[:NSTRIPE]
    isems = scratch[NSTRIPE:2 * NSTRIPE]
    osems = scratch[2 * NSTRIPE:]
    in_cp = []
    out_cp = []
    for i in range(NSTRIPE):
        sl = pl.ds(i * STRIPE, STRIPE)
        in_cp.append(pltpu.make_async_copy(w_hbm.at[sl, :], bufs[i], isems[i]))
        out_cp.append(pltpu.make_async_copy(bufs[i], o_hbm.at[sl, :], osems[i]))
    for c in in_cp:
        c.start()
    for i in range(NSTRIPE):
        in_cp[i].wait()
        out_cp[i].start()
    for c in out_cp:
        c.wait()


def kernel(x, W):
    del x
    Wt = W.T
    out_t = pl.pallas_call(
        _dma_copy_kernel,
        in_specs=[pl.BlockSpec(memory_space=pltpu.MemorySpace.HBM)],
        out_specs=pl.BlockSpec(memory_space=pltpu.MemorySpace.HBM),
        out_shape=jax.ShapeDtypeStruct((EMBED, NUM_ROWS), jnp.float32),
        scratch_shapes=(
            [pltpu.VMEM((STRIPE, NUM_ROWS), jnp.float32)] * NSTRIPE
            + [pltpu.SemaphoreType.DMA] * (2 * NSTRIPE)
        ),
    )(Wt)
    return out_t.T


# speedup vs baseline: 6.7158x; 1.0123x over previous
import jax
import jax.numpy as jnp
from jax.experimental import pallas as pl
from jax.experimental.pallas import tpu as pltpu


NUM_ROWS = 100000
EMBED = 64
NSTRIPE = 4
STRIPE = EMBED // NSTRIPE


def _dma_copy_kernel(w_hbm, o_hbm, *scratch):
    bufs = scratch[:NSTRIPE]
    isems = scratch[NSTRIPE:2 * NSTRIPE]
    osems = scratch[2 * NSTRIPE:]
    in_cp = []
    out_cp = []
    for i in range(NSTRIPE):
        sl = pl.ds(i * STRIPE, STRIPE)
        in_cp.append(pltpu.make_async_copy(w_hbm.at[sl, :], bufs[i], isems[i]))
        out_cp.append(pltpu.make_async_copy(bufs[i], o_hbm.at[sl, :], osems[i]))
    for c in in_cp:
        c.start()
    for i in range(NSTRIPE):
        in_cp[i].wait()
        out_cp[i].start()
    for c in out_cp:
        c.wait()


def kernel(x, W):
    del x
    Wt = W.T
    out_t = pl.pallas_call(
        _dma_copy_kernel,
        in_specs=[pl.BlockSpec(memory_space=pltpu.MemorySpace.HBM)],
        out_specs=pl.BlockSpec(memory_space=pltpu.MemorySpace.HBM),
        out_shape=jax.ShapeDtypeStruct((EMBED, NUM_ROWS), jnp.float32),
        scratch_shapes=(
            [pltpu.VMEM((STRIPE, NUM_ROWS), jnp.float32)] * NSTRIPE
            + [pltpu.SemaphoreType.DMA] * (2 * NSTRIPE)
        ),
    )(Wt)
    return out_t.T


# speedup vs baseline: 6.9235x; 1.0309x over previous
import jax
import jax.numpy as jnp
from jax.experimental import pallas as pl
from jax.experimental.pallas import tpu as pltpu


NUM_ROWS = 100000
EMBED = 64
BLOCK_SUB = 32


def _copy_kernel(w_ref, o_ref):
    o_ref[...] = w_ref[...]


def kernel(x, W):
    del x
    Wt = W.T
    out_t = pl.pallas_call(
        _copy_kernel,
        grid=(EMBED // BLOCK_SUB,),
        in_specs=[pl.BlockSpec((BLOCK_SUB, NUM_ROWS), lambda i: (i, 0))],
        out_specs=pl.BlockSpec((BLOCK_SUB, NUM_ROWS), lambda i: (i, 0)),
        out_shape=jax.ShapeDtypeStruct((EMBED, NUM_ROWS), jnp.float32),
    )(Wt)
    return out_t.T
